# Initial kernel scaffold; baseline (speedup 1.0000x reference)
import jax
import jax.numpy as jnp
from jax.experimental import pallas as pl


def kernel(x, mask, table, W, b):
    raise NotImplementedError("write your pallas kernel here")


# speedup vs baseline: 2.5663x; 2.5663x over previous
import functools

import jax
import jax.numpy as jnp
from jax import lax
from jax.experimental import pallas as pl
from jax.experimental.pallas import tpu as pltpu
from jax.experimental.pallas import tpu_sc as plsc

_BATCH = 16384
_SEQ = 50
_DIM = 32

_NC = 2
_NS = 16
_NW = _NC * _NS
_BPW = _BATCH // _NW
_RPC = 2
_CW = _RPC * _SEQ
_NCHUNK = _BPW // _RPC


def _pool_body(x_hbm, m_hbm, table_hbm, out_hbm,
               idx_v, mask_v, rows0, rows1, pooled_v, sem0, sem1):
    wid = lax.axis_index("s") * _NC + lax.axis_index("c")

    pltpu.sync_copy(x_hbm.at[wid], idx_v)
    pltpu.sync_copy(m_hbm.at[wid], mask_v)

    def start(c, buf, sem):
        pltpu.async_copy(table_hbm.at[idx_v.at[c]], buf, sem)

    def wait(c, buf, sem):
        pltpu.make_async_copy(table_hbm.at[idx_v.at[c]], buf, sem).wait()

    def compute(c, buf):
        for r in range(_RPC):
            base = r * _SEQ
            mv = [mask_v[c, pl.ds(base, 16)],
                  mask_v[c, pl.ds(base + 16, 16)],
                  mask_v[c, pl.ds(base + 32, 16)],
                  mask_v[c, pl.ds(base + 34, 16)]]
            acc = [jnp.zeros((16,), jnp.float32) for _ in range(4)]
            for l in range(_SEQ):
                if l < 48:
                    m = mv[l // 16][l % 16]
                else:
                    m = mv[3][l - 34]
                acc[l % 2] = acc[l % 2] + m * buf[base + l, 0:16]
                acc[2 + l % 2] = acc[2 + l % 2] + m * buf[base + l, 16:32]
            row = _RPC * c + r
            pooled_v[row, 0:16] = acc[0] + acc[1]
            pooled_v[row, 16:32] = acc[2] + acc[3]

    start(0, rows0, sem0)
    start(1, rows1, sem1)

    def step(i, _):
        jj = 2 * i
        wait(jj, rows0, sem0)
        compute(jj, rows0)

        @pl.when(jj + 2 < _NCHUNK)
        def _():
            start(jj + 2, rows0, sem0)

        wait(jj + 1, rows1, sem1)
        compute(jj + 1, rows1)

        @pl.when(jj + 3 < _NCHUNK)
        def _():
            start(jj + 3, rows1, sem1)

        return _

    lax.fori_loop(0, _NCHUNK // 2, step, None)

    pltpu.sync_copy(pooled_v, out_hbm.at[pl.ds(wid * _BPW, _BPW)])


@functools.partial(
    pl.kernel,
    out_type=jax.ShapeDtypeStruct((_BATCH, _DIM), jnp.float32),
    mesh=plsc.VectorSubcoreMesh(core_axis_name="c", subcore_axis_name="s"),
    compiler_params=pltpu.CompilerParams(use_tc_tiling_on_sc=False),
    scratch_types=[
        pltpu.VMEM((_NCHUNK, _CW), jnp.int32),
        pltpu.VMEM((_NCHUNK, _CW), jnp.float32),
        pltpu.VMEM((_CW, _DIM), jnp.float32),
        pltpu.VMEM((_CW, _DIM), jnp.float32),
        pltpu.VMEM((_BPW, _DIM), jnp.float32),
        pltpu.SemaphoreType.DMA,
        pltpu.SemaphoreType.DMA,
    ],
)
def _pool(x_hbm, m_hbm, table_hbm, out_hbm,
          idx_v, mask_v, rows0, rows1, pooled_v, sem0, sem1):
    _pool_body(x_hbm, m_hbm, table_hbm, out_hbm,
               idx_v, mask_v, rows0, rows1, pooled_v, sem0, sem1)


_MM_BLK = 2048


def _mm_body(s_ref, wt_ref, b_ref, o_ref):
    acc = jnp.dot(s_ref[...], wt_ref[...], preferred_element_type=jnp.float32)
    o_ref[...] = acc * (1.0 / _SEQ) + b_ref[...]


def _linear(s, wt, b2):
    return pl.pallas_call(
        _mm_body,
        out_shape=jax.ShapeDtypeStruct((_BATCH, _DIM), jnp.float32),
        grid=(_BATCH // _MM_BLK,),
        in_specs=[
            pl.BlockSpec((_MM_BLK, _DIM), lambda i: (i, 0)),
            pl.BlockSpec((_DIM, _DIM), lambda i: (0, 0)),
            pl.BlockSpec((1, _DIM), lambda i: (0, 0)),
        ],
        out_specs=pl.BlockSpec((_MM_BLK, _DIM), lambda i: (i, 0)),
    )(s, wt, b2)


@jax.jit
def kernel(x, mask, table, W, b):
    xi = x.astype(jnp.int32).reshape(_NW, _NCHUNK, _CW)
    mi = mask.astype(jnp.float32).reshape(_NW, _NCHUNK, _CW)
    pooled = _pool(xi, mi, table)
    return _linear(pooled, W.T, b.reshape(1, _DIM))


# speedup vs baseline: 2.7877x; 1.0863x over previous
import functools

import jax
import jax.numpy as jnp
from jax import lax
from jax.experimental import pallas as pl
from jax.experimental.pallas import tpu as pltpu
from jax.experimental.pallas import tpu_sc as plsc

_BATCH = 16384
_SEQ = 50
_DIM = 32

_NC = 2
_NS = 16
_NW = _NC * _NS
_BPW = _BATCH // _NW
_RPC = 2
_CW = _RPC * _SEQ
_NCHUNK = _BPW // _RPC


_NBUF = 4


def _pool_body(x_hbm, m_hbm, table_hbm, out_hbm,
               idx_v, mask_v, rows0, rows1, rows2, rows3, pooled_v,
               sem0, sem1, sem2, sem3):
    bufs = (rows0, rows1, rows2, rows3)
    sems = (sem0, sem1, sem2, sem3)
    wid = lax.axis_index("s") * _NC + lax.axis_index("c")

    pltpu.sync_copy(x_hbm.at[wid], idx_v)
    pltpu.sync_copy(m_hbm.at[wid], mask_v)

    def start(c, buf, sem):
        pltpu.async_copy(table_hbm.at[idx_v.at[c]], buf, sem)

    def wait(c, buf, sem):
        pltpu.make_async_copy(table_hbm.at[idx_v.at[c]], buf, sem).wait()

    def compute(c, buf):
        for r in range(_RPC):
            base = r * _SEQ
            mv = [mask_v[c, pl.ds(base, 16)],
                  mask_v[c, pl.ds(base + 16, 16)],
                  mask_v[c, pl.ds(base + 32, 16)],
                  mask_v[c, pl.ds(base + 34, 16)]]
            acc = [jnp.zeros((16,), jnp.float32) for _ in range(4)]
            for l in range(_SEQ):
                if l < 48:
                    m = mv[l // 16][l % 16]
                else:
                    m = mv[3][l - 34]
                acc[l % 2] = acc[l % 2] + m * buf[base + l, 0:16]
                acc[2 + l % 2] = acc[2 + l % 2] + m * buf[base + l, 16:32]
            row = _RPC * c + r
            pooled_v[row, 0:16] = acc[0] + acc[1]
            pooled_v[row, 16:32] = acc[2] + acc[3]

    for k in range(_NBUF):
        start(k, bufs[k], sems[k])

    def step(i, _):
        jj = _NBUF * i
        for r in range(_NBUF):
            j = jj + r
            wait(j, bufs[r], sems[r])
            compute(j, bufs[r])

            @pl.when(j + _NBUF < _NCHUNK)
            def _():
                start(j + _NBUF, bufs[r], sems[r])

        return _

    lax.fori_loop(0, _NCHUNK // _NBUF, step, None)

    pltpu.sync_copy(pooled_v, out_hbm.at[pl.ds(wid * _BPW, _BPW)])


@functools.partial(
    pl.kernel,
    out_type=jax.ShapeDtypeStruct((_BATCH, _DIM), jnp.float32),
    mesh=plsc.VectorSubcoreMesh(core_axis_name="c", subcore_axis_name="s"),
    compiler_params=pltpu.CompilerParams(use_tc_tiling_on_sc=False),
    scratch_types=[
        pltpu.VMEM((_NCHUNK, _CW), jnp.int32),
        pltpu.VMEM((_NCHUNK, _CW), jnp.float32),
        pltpu.VMEM((_CW, _DIM), jnp.float32),
        pltpu.VMEM((_CW, _DIM), jnp.float32),
        pltpu.VMEM((_CW, _DIM), jnp.float32),
        pltpu.VMEM((_CW, _DIM), jnp.float32),
        pltpu.VMEM((_BPW, _DIM), jnp.float32),
        pltpu.SemaphoreType.DMA,
        pltpu.SemaphoreType.DMA,
        pltpu.SemaphoreType.DMA,
        pltpu.SemaphoreType.DMA,
    ],
)
def _pool(x_hbm, m_hbm, table_hbm, out_hbm,
          idx_v, mask_v, rows0, rows1, rows2, rows3, pooled_v,
          sem0, sem1, sem2, sem3):
    _pool_body(x_hbm, m_hbm, table_hbm, out_hbm,
               idx_v, mask_v, rows0, rows1, rows2, rows3, pooled_v,
               sem0, sem1, sem2, sem3)


_MM_BLK = 2048


def _mm_body(s_ref, wt_ref, b_ref, o_ref):
    acc = jnp.dot(s_ref[...], wt_ref[...], preferred_element_type=jnp.float32)
    o_ref[...] = acc * (1.0 / _SEQ) + b_ref[...]


def _linear(s, wt, b2):
    return pl.pallas_call(
        _mm_body,
        out_shape=jax.ShapeDtypeStruct((_BATCH, _DIM), jnp.float32),
        grid=(_BATCH // _MM_BLK,),
        in_specs=[
            pl.BlockSpec((_MM_BLK, _DIM), lambda i: (i, 0)),
            pl.BlockSpec((_DIM, _DIM), lambda i: (0, 0)),
            pl.BlockSpec((1, _DIM), lambda i: (0, 0)),
        ],
        out_specs=pl.BlockSpec((_MM_BLK, _DIM), lambda i: (i, 0)),
    )(s, wt, b2)


@jax.jit
def kernel(x, mask, table, W, b):
    xi = x.astype(jnp.int32).reshape(_NW, _NCHUNK, _CW)
    mi = mask.astype(jnp.float32).reshape(_NW, _NCHUNK, _CW)
    pooled = _pool(xi, mi, table)
    return _linear(pooled, W.T, b.reshape(1, _DIM))
